# baseline (speedup 1.0000x reference)
import functools
import numpy as np
import jax
import jax.numpy as jnp
from jax import lax
from jax.experimental import pallas as pl
from jax.experimental.pallas import tpu as pltpu
from jax.experimental.pallas import tpu_sc as plsc

_C = 96
_K = 9
_G = 4
_EPS = 1e-05
_B, _H, _W = 16, 32, 32
_N = _H * _W
_R = 256
_CP = 128
_PREC = jax.lax.Precision.HIGHEST
_PLOW = jax.lax.Precision.DEFAULT

_NW = 32
_NPW = _B * _N // _NW
_SUP = 8
_SUBC = 8
_CN = 8
_CI = _CN * _K


def _gelu(x):
    return 0.5 * x * (1.0 + jax.lax.erf(x * np.float32(2.0 ** -0.5)))


def _feat_body(xt_ref, w1_ref, s1_ref, t1_ref, h_ref, xn_ref, sq_ref):
    xt = xt_ref[0]
    h = jnp.dot(xt, w1_ref[...], preferred_element_type=jnp.float32,
                precision=_PREC) * s1_ref[...] + t1_ref[...]
    nrm = jnp.sqrt(jnp.sum(h * h, axis=1, keepdims=True))
    xn = h / jnp.maximum(nrm, 1e-12)
    h_ref[0] = h
    xn_ref[0] = xn
    ones_row = jnp.ones((1, _CP), jnp.float32)
    sq_ref[0] = jax.lax.dot_general(ones_row, xn * xn, (((1,), (1,)), ((), ())),
                                    preferred_element_type=jnp.float32,
                                    precision=_PREC)


def _topk_body(xn_full_ref, sq_row_ref, xn_tile_ref, rp_ref, idx_ref):
    xn = xn_full_ref[0]
    sq_row = sq_row_ref[0]
    xn_t = xn_tile_ref[0]
    rp = rp_ref[0]
    b_off = pl.program_id(1) * _N

    sq_t = jnp.sum(xn_t * xn_t, axis=1, keepdims=True)
    inner = jax.lax.dot_general(xn_t, xn, (((1,), (1,)), ((), ())),
                                preferred_element_type=jnp.float32,
                                precision=_PREC)
    d = sq_t + (-2.0) * inner + sq_row + rp

    col = jax.lax.broadcasted_iota(jnp.int32, (_R, _N), 1)
    u = jax.lax.bitcast_convert_type(jnp.maximum(d + 16.0, 1.0), jnp.int32)
    keys = jax.lax.bitcast_convert_type((u & jnp.int32(-1024)) | col,
                                        jnp.float32)
    for k in range(_K):
        kmin = jnp.min(keys, axis=1, keepdims=True)
        ik = (jax.lax.bitcast_convert_type(kmin, jnp.int32)
              & jnp.int32(1023)) + b_off
        idx_ref[0, :, pl.ds(k, 1)] = ik
        if k < _K - 1:
            keys = jnp.where(keys == kmin, jnp.float32(3.0e38), keys)


def _sc_gather_body(h_hbm, idx_hbm, out_hbm, idx_v, rows_v, out_v, sem):
    wid = lax.axis_index("s") * 2 + lax.axis_index("c")
    base = wid * _NPW
    pltpu.sync_copy(idx_hbm.at[pl.ds(base * _K, _NPW * _K)], idx_v)

    def super_body(si, carry):
        handles = [
            pltpu.async_copy(
                h_hbm.at[idx_v.at[pl.ds(si * (_SUBC * _CI) + j * _CI, _CI)]],
                rows_v.at[pl.ds(j * _CI, _CI)], sem)
            for j in range(_SUBC)
        ]
        for hnd in handles:
            hnd.wait()

        def node_body(n2, carry2):
            r0 = n2 * _K
            for v in range(_CP // 16):
                sl = pl.ds(v * 16, 16)
                acc = rows_v[r0, sl]
                for k in range(1, _K):
                    acc = jnp.maximum(acc, rows_v[r0 + k, sl])
                out_v[n2, sl] = acc
            return carry2

        lax.fori_loop(0, _SUBC * _CN, node_body, 0)
        pltpu.sync_copy(out_v,
                        out_hbm.at[pl.ds(base + si * (_SUBC * _CN),
                                         _SUBC * _CN)])
        return carry

    lax.fori_loop(0, _SUP, super_body, 0)


def _tail_body(h_tile_ref, mg_tile_ref, xt_tile_ref,
               a1_ref, a2_ref, s2_ref, t2_ref,
               g2_ref, s3_ref, t3_ref,
               f1_ref, s4_ref, t4_ref,
               f2_ref, s5_ref, t5_ref,
               out_ref):
    h_t = h_tile_ref[0]
    xjm = mg_tile_ref[0] - h_t
    xt_t = xt_tile_ref[0]

    mr = (jnp.dot(h_t, a1_ref[...], preferred_element_type=jnp.float32,
                  precision=_PLOW)
          + jnp.dot(xjm, a2_ref[...], preferred_element_type=jnp.float32,
                    precision=_PLOW))
    mr = _gelu(mr * s2_ref[...] + t2_ref[...])

    g = jnp.dot(mr, g2_ref[...], preferred_element_type=jnp.float32,
                precision=_PLOW) * s3_ref[...] + t3_ref[...]
    score = g + xt_t

    f = _gelu(jnp.dot(score, f1_ref[...], preferred_element_type=jnp.float32,
                      precision=_PLOW) * s4_ref[...] + t4_ref[...])
    f = jnp.dot(f, f2_ref[...], preferred_element_type=jnp.float32,
                precision=_PLOW) * s5_ref[...] + t5_ref[...]
    out_ref[0] = f + score


def _sc_gather_max(h2, gidx_flat):
    mesh = plsc.VectorSubcoreMesh(core_axis_name="c", subcore_axis_name="s")
    fn = functools.partial(
        pl.kernel, mesh=mesh,
        out_type=jax.ShapeDtypeStruct((_B * _N, _CP), jnp.float32),
        scratch_types=[
            pltpu.VMEM((_NPW * _K,), jnp.int32),
            pltpu.VMEM((_SUBC * _CI, _CP), jnp.float32),
            pltpu.VMEM((_SUBC * _CN, _CP), jnp.float32),
            pltpu.SemaphoreType.DMA,
        ],
    )(_sc_gather_body)
    return fn(h2, gidx_flat)


def kernel(x, g_fc1_w, g_fc1_b, g_fc1_gamma, g_fc1_beta, mr_w, mr_b,
           mr_gamma, mr_beta, g_fc2_w, g_fc2_b, g_fc2_gamma, g_fc2_beta,
           f_fc1_w, f_fc1_b, f_fc1_gamma, f_fc1_beta, f_fc2_w, f_fc2_b,
           f_fc2_gamma, f_fc2_beta, rel_pos):
    inv = np.float32(1.0 / np.sqrt(1.0 + _EPS))

    def fold(w_b, gamma, beta):
        s = gamma * inv
        return s[None, :], (w_b * s + beta)[None, :]

    s1, t1 = fold(g_fc1_b, g_fc1_gamma, g_fc1_beta)
    s2i, t2i = fold(mr_b, mr_gamma, mr_beta)
    s3, t3 = fold(g_fc2_b, g_fc2_gamma, g_fc2_beta)
    s4, t4 = fold(f_fc1_b, f_fc1_gamma, f_fc1_beta)
    s5, t5 = fold(f_fc2_b, f_fc2_gamma, f_fc2_beta)

    q = np.concatenate([2 * np.arange(_C), 2 * np.arange(_C) + 1])
    wg = mr_w.reshape(_G, 2 * _C // _G, 2 * _C // _G)
    w2 = jax.scipy.linalg.block_diag(*[wg[g] for g in range(_G)])
    a = w2[q][:, q]
    a1 = a[:, :_C].T
    a2 = a[:, _C:].T
    s2 = s2i[:, q]
    t2 = t2i[:, q]
    g2 = g_fc2_w[:, q].T

    pad = [(0, 0), (0, _CP - _C)]
    w1 = jnp.pad(g_fc1_w.T, pad)
    s1 = jnp.pad(s1, pad)
    t1 = jnp.pad(t1, pad)
    a1 = jnp.pad(a1, [(0, _CP - _C), (0, 0)])
    a2 = jnp.pad(a2, [(0, _CP - _C), (0, 0)])
    f1 = f_fc1_w.T
    f2 = f_fc2_w.T

    xt = jnp.transpose(x.reshape(_B, _C, _N), (0, 2, 1))

    h_all, xn_all, sq_all = pl.pallas_call(
        _feat_body,
        grid=(_B,),
        in_specs=[
            pl.BlockSpec((1, _N, _C), lambda b: (b, 0, 0)),
            pl.BlockSpec((_C, _CP), lambda b: (0, 0)),
            pl.BlockSpec((1, _CP), lambda b: (0, 0)),
            pl.BlockSpec((1, _CP), lambda b: (0, 0)),
        ],
        out_specs=[
            pl.BlockSpec((1, _N, _CP), lambda b: (b, 0, 0)),
            pl.BlockSpec((1, _N, _CP), lambda b: (b, 0, 0)),
            pl.BlockSpec((1, 1, _N), lambda b: (b, 0, 0)),
        ],
        out_shape=[
            jax.ShapeDtypeStruct((_B, _N, _CP), jnp.float32),
            jax.ShapeDtypeStruct((_B, _N, _CP), jnp.float32),
            jax.ShapeDtypeStruct((_B, 1, _N), jnp.float32),
        ],
        compiler_params=pltpu.CompilerParams(
            dimension_semantics=("arbitrary",),
        ),
    )(xt, w1, s1, t1)

    nt = _N // _R
    grid = (nt, _B)

    def full_img(t, b):
        return (b, 0, 0)

    def row_tile(t, b):
        return (b, t, 0)

    def rp_tile(t, b):
        return (0, t, 0)

    def w_map(t, b):
        return (0, 0)

    wspec = lambda shape: pl.BlockSpec(shape, w_map)

    gidx = pl.pallas_call(
        _topk_body,
        grid=grid,
        in_specs=[
            pl.BlockSpec((1, _N, _CP), full_img),
            pl.BlockSpec((1, 1, _N), full_img),
            pl.BlockSpec((1, _R, _CP), row_tile),
            pl.BlockSpec((1, _R, _N), rp_tile),
        ],
        out_specs=pl.BlockSpec((1, _R, _K), row_tile),
        out_shape=jax.ShapeDtypeStruct((_B, _N, _K), jnp.int32),
        compiler_params=pltpu.CompilerParams(
            dimension_semantics=("arbitrary", "arbitrary"),
        ),
    )(xn_all, sq_all, xn_all, rel_pos)

    h2 = h_all.reshape(_B * _N, _CP)
    mg = _sc_gather_max(h2, gidx.reshape(-1)).reshape(_B, _N, _CP)

    out = pl.pallas_call(
        _tail_body,
        grid=grid,
        in_specs=[
            pl.BlockSpec((1, _R, _CP), row_tile),
            pl.BlockSpec((1, _R, _CP), row_tile),
            pl.BlockSpec((1, _R, _C), row_tile),
            wspec((_CP, 2 * _C)), wspec((_CP, 2 * _C)),
            wspec((1, 2 * _C)), wspec((1, 2 * _C)),
            wspec((2 * _C, _C)),
            wspec((1, _C)), wspec((1, _C)),
            wspec((_C, 4 * _C)),
            wspec((1, 4 * _C)), wspec((1, 4 * _C)),
            wspec((4 * _C, _C)),
            wspec((1, _C)), wspec((1, _C)),
        ],
        out_specs=pl.BlockSpec((1, _R, _C), row_tile),
        out_shape=jax.ShapeDtypeStruct((_B, _N, _C), jnp.float32),
        compiler_params=pltpu.CompilerParams(
            dimension_semantics=("arbitrary", "arbitrary"),
        ),
    )(h_all, mg, xt,
      a1, a2, s2, t2, g2, s3, t3, f1, s4, t4, f2, s5, t5)

    return jnp.transpose(out, (0, 2, 1)).reshape(_B, _C, _H, _W)


# speedup vs baseline: 1.1674x; 1.1674x over previous
import functools
import numpy as np
import jax
import jax.numpy as jnp
from jax import lax
from jax.experimental import pallas as pl
from jax.experimental.pallas import tpu as pltpu
from jax.experimental.pallas import tpu_sc as plsc

_C = 96
_K = 9
_G = 4
_EPS = 1e-05
_B, _H, _W = 16, 32, 32
_N = _H * _W
_R = 256
_CP = 128
_PREC = jax.lax.Precision.HIGHEST
_PLOW = jax.lax.Precision.DEFAULT

_HALVES = 2
_BH = _B // _HALVES
_KG = _K - 1
_NW = 32
_NPW = _BH * _N // _NW
_CN = 16
_NCHUNK = _NPW // _CN
_NBUF = 4


def _gelu(x):
    return 0.5 * x * (1.0 + jax.lax.erf(x * np.float32(2.0 ** -0.5)))


def _feat_body(xt_ref, w1_ref, s1_ref, t1_ref, h_ref, xn_ref, sq_ref):
    xt = xt_ref[0]
    h = jnp.dot(xt, w1_ref[...], preferred_element_type=jnp.float32,
                precision=_PREC) * s1_ref[...] + t1_ref[...]
    nrm = jnp.sqrt(jnp.sum(h * h, axis=1, keepdims=True))
    xn = h / jnp.maximum(nrm, 1e-12)
    h_ref[0] = h
    xn_ref[0] = xn
    ones_row = jnp.ones((1, _CP), jnp.float32)
    sq_ref[0] = jax.lax.dot_general(ones_row, xn * xn, (((1,), (1,)), ((), ())),
                                    preferred_element_type=jnp.float32,
                                    precision=_PREC)


def _make_topk_body(b0):
    def _topk_body(xn_full_ref, sq_row_ref, xn_tile_ref, rp_ref, idx_ref):
        xn = xn_full_ref[0]
        sq_row = sq_row_ref[0]
        xn_t = xn_tile_ref[0]
        rp = rp_ref[0]
        b_off = (pl.program_id(1) + b0) * _N

        sq_t = jnp.sum(xn_t * xn_t, axis=1, keepdims=True)
        inner = jax.lax.dot_general(xn_t, xn, (((1,), (1,)), ((), ())),
                                    preferred_element_type=jnp.float32,
                                    precision=_PREC)
        d = sq_t + (-2.0) * inner + sq_row + rp

        col = jax.lax.broadcasted_iota(jnp.int32, (_R, _N), 1)
        u = jax.lax.bitcast_convert_type(jnp.maximum(d + 16.0, 1.0), jnp.int32)
        keys = jax.lax.bitcast_convert_type((u & jnp.int32(-1024)) | col,
                                            jnp.float32)
        for k in range(_K):
            kmin = jnp.min(keys, axis=1, keepdims=True)
            if k > 0:
                ik = (jax.lax.bitcast_convert_type(kmin, jnp.int32)
                      & jnp.int32(1023)) + b_off
                idx_ref[0, :, pl.ds(k - 1, 1)] = ik
            if k < _K - 1:
                keys = jnp.where(keys == kmin, jnp.float32(3.0e38), keys)
    return _topk_body


def _sc_gather_body(h_hbm, idx_hbm, out_hbm, idx_v, rows_v, out_v, sem):
    wid = lax.axis_index("s") * 2 + lax.axis_index("c")
    base = wid * _NPW
    pltpu.sync_copy(idx_hbm.at[pl.ds(base * _KG, _NPW * _KG)], idx_v)

    def issue(ci, b):
        return pltpu.async_copy(
            h_hbm.at[idx_v.at[pl.ds(ci * (_CN * _KG), _CN * _KG)]],
            rows_v.at[pl.ds(b * (_CN * _KG), _CN * _KG)], sem)

    handles = [None] * _NCHUNK
    for b in range(_NBUF):
        handles[b] = issue(b, b)
    for ci in range(_NCHUNK):
        handles[ci].wait()
        b = ci % _NBUF

        def node_body(n2, carry, _b=b, _ci=ci):
            r0 = _b * (_CN * _KG) + n2 * _KG
            for v in range(_C // 16):
                sl = pl.ds(v * 16, 16)
                acc = rows_v[r0, sl]
                for k in range(1, _KG):
                    acc = jnp.maximum(acc, rows_v[r0 + k, sl])
                out_v[_ci * _CN + n2, sl] = acc
            return carry

        lax.fori_loop(0, _CN, node_body, 0)
        nxt = ci + _NBUF
        if nxt < _NCHUNK:
            handles[nxt] = issue(nxt, b)
    pltpu.sync_copy(out_v, out_hbm.at[pl.ds(base, _NPW)])


def _sc_gather_max(h2, gidx_flat):
    mesh = plsc.VectorSubcoreMesh(core_axis_name="c", subcore_axis_name="s")
    fn = functools.partial(
        pl.kernel, mesh=mesh,
        out_type=jax.ShapeDtypeStruct((_BH * _N, _C), jnp.float32),
        scratch_types=[
            pltpu.VMEM((_NPW * _KG,), jnp.int32),
            pltpu.VMEM((_NBUF * _CN * _KG, _CP), jnp.float32),
            pltpu.VMEM((_NPW, _C), jnp.float32),
            pltpu.SemaphoreType.DMA,
        ],
    )(_sc_gather_body)
    return fn(h2, gidx_flat)


def _tail_body(h_tile_ref, mg_tile_ref, xt_tile_ref,
               a1_ref, a2_ref, s2_ref, t2_ref,
               g2_ref, s3_ref, t3_ref,
               f1_ref, s4_ref, t4_ref,
               f2_ref, s5_ref, t5_ref,
               out_ref):
    h_t = h_tile_ref[0][:, :_C]
    mg = jnp.maximum(mg_tile_ref[0], h_t)
    xjm = mg - h_t
    xt_t = xt_tile_ref[0]

    mr = (jnp.dot(h_t, a1_ref[...], preferred_element_type=jnp.float32,
                  precision=_PLOW)
          + jnp.dot(xjm, a2_ref[...], preferred_element_type=jnp.float32,
                    precision=_PLOW))
    mr = _gelu(mr * s2_ref[...] + t2_ref[...])

    g = jnp.dot(mr, g2_ref[...], preferred_element_type=jnp.float32,
                precision=_PLOW) * s3_ref[...] + t3_ref[...]
    score = g + xt_t

    f = _gelu(jnp.dot(score, f1_ref[...], preferred_element_type=jnp.float32,
                      precision=_PLOW) * s4_ref[...] + t4_ref[...])
    f = jnp.dot(f, f2_ref[...], preferred_element_type=jnp.float32,
                precision=_PLOW) * s5_ref[...] + t5_ref[...]
    out_ref[0] = f + score


def kernel(x, g_fc1_w, g_fc1_b, g_fc1_gamma, g_fc1_beta, mr_w, mr_b,
           mr_gamma, mr_beta, g_fc2_w, g_fc2_b, g_fc2_gamma, g_fc2_beta,
           f_fc1_w, f_fc1_b, f_fc1_gamma, f_fc1_beta, f_fc2_w, f_fc2_b,
           f_fc2_gamma, f_fc2_beta, rel_pos):
    inv = np.float32(1.0 / np.sqrt(1.0 + _EPS))

    def fold(w_b, gamma, beta):
        s = gamma * inv
        return s[None, :], (w_b * s + beta)[None, :]

    s1, t1 = fold(g_fc1_b, g_fc1_gamma, g_fc1_beta)
    s2i, t2i = fold(mr_b, mr_gamma, mr_beta)
    s3, t3 = fold(g_fc2_b, g_fc2_gamma, g_fc2_beta)
    s4, t4 = fold(f_fc1_b, f_fc1_gamma, f_fc1_beta)
    s5, t5 = fold(f_fc2_b, f_fc2_gamma, f_fc2_beta)

    q = np.concatenate([2 * np.arange(_C), 2 * np.arange(_C) + 1])
    wg = mr_w.reshape(_G, 2 * _C // _G, 2 * _C // _G)
    w2 = jax.scipy.linalg.block_diag(*[wg[g] for g in range(_G)])
    a = w2[q][:, q]
    a1 = a[:, :_C].T
    a2 = a[:, _C:].T
    s2 = s2i[:, q]
    t2 = t2i[:, q]
    g2 = g_fc2_w[:, q].T

    pad = [(0, 0), (0, _CP - _C)]
    w1 = jnp.pad(g_fc1_w.T, pad)
    s1 = jnp.pad(s1, pad)
    t1 = jnp.pad(t1, pad)
    f1 = f_fc1_w.T
    f2 = f_fc2_w.T

    xt = jnp.transpose(x.reshape(_B, _C, _N), (0, 2, 1))

    h_all, xn_all, sq_all = pl.pallas_call(
        _feat_body,
        grid=(_B,),
        in_specs=[
            pl.BlockSpec((1, _N, _C), lambda b: (b, 0, 0)),
            pl.BlockSpec((_C, _CP), lambda b: (0, 0)),
            pl.BlockSpec((1, _CP), lambda b: (0, 0)),
            pl.BlockSpec((1, _CP), lambda b: (0, 0)),
        ],
        out_specs=[
            pl.BlockSpec((1, _N, _CP), lambda b: (b, 0, 0)),
            pl.BlockSpec((1, _N, _CP), lambda b: (b, 0, 0)),
            pl.BlockSpec((1, 1, _N), lambda b: (b, 0, 0)),
        ],
        out_shape=[
            jax.ShapeDtypeStruct((_B, _N, _CP), jnp.float32),
            jax.ShapeDtypeStruct((_B, _N, _CP), jnp.float32),
            jax.ShapeDtypeStruct((_B, 1, _N), jnp.float32),
        ],
        compiler_params=pltpu.CompilerParams(
            dimension_semantics=("arbitrary",),
        ),
    )(xt, w1, s1, t1)

    h2 = h_all.reshape(_B * _N, _CP)
    nt = _N // _R
    grid = (nt, _BH)

    def rp_tile(t, b):
        return (0, t, 0)

    def w_map(t, b):
        return (0, 0)

    wspec = lambda shape: pl.BlockSpec(shape, w_map)

    gidx_halves = []
    for hf in range(_HALVES):
        b0 = hf * _BH
        gidx_halves.append(pl.pallas_call(
            _make_topk_body(b0),
            grid=grid,
            in_specs=[
                pl.BlockSpec((1, _N, _CP), lambda t, b, b0=b0: (b + b0, 0, 0)),
                pl.BlockSpec((1, 1, _N), lambda t, b, b0=b0: (b + b0, 0, 0)),
                pl.BlockSpec((1, _R, _CP), lambda t, b, b0=b0: (b + b0, t, 0)),
                pl.BlockSpec((1, _R, _N), rp_tile),
            ],
            out_specs=pl.BlockSpec((1, _R, _KG), lambda t, b: (b, t, 0)),
            out_shape=jax.ShapeDtypeStruct((_BH, _N, _KG), jnp.int32),
            compiler_params=pltpu.CompilerParams(
                dimension_semantics=("arbitrary", "arbitrary"),
            ),
        )(xn_all, sq_all, xn_all, rel_pos))

    mg_halves = [
        _sc_gather_max(h2, gidx_halves[hf].reshape(-1)).reshape(_BH, _N, _C)
        for hf in range(_HALVES)
    ]

    out_halves = []
    for hf in range(_HALVES):
        b0 = hf * _BH
        out_halves.append(pl.pallas_call(
            _tail_body,
            grid=grid,
            in_specs=[
                pl.BlockSpec((1, _R, _CP), lambda t, b, b0=b0: (b + b0, t, 0)),
                pl.BlockSpec((1, _R, _C), lambda t, b: (b, t, 0)),
                pl.BlockSpec((1, _R, _C), lambda t, b, b0=b0: (b + b0, t, 0)),
                wspec((_C, 2 * _C)), wspec((_C, 2 * _C)),
                wspec((1, 2 * _C)), wspec((1, 2 * _C)),
                wspec((2 * _C, _C)),
                wspec((1, _C)), wspec((1, _C)),
                wspec((_C, 4 * _C)),
                wspec((1, 4 * _C)), wspec((1, 4 * _C)),
                wspec((4 * _C, _C)),
                wspec((1, _C)), wspec((1, _C)),
            ],
            out_specs=pl.BlockSpec((1, _R, _C), lambda t, b: (b, t, 0)),
            out_shape=jax.ShapeDtypeStruct((_BH, _N, _C), jnp.float32),
            compiler_params=pltpu.CompilerParams(
                dimension_semantics=("arbitrary", "arbitrary"),
            ),
        )(h_all, mg_halves[hf], xt,
          a1, a2, s2, t2, g2, s3, t3, f1, s4, t4, f2, s5, t5))

    out = jnp.concatenate(out_halves, axis=0)
    return jnp.transpose(out, (0, 2, 1)).reshape(_B, _C, _H, _W)
